# baseline (speedup 1.0000x reference)
import functools

import jax
import jax.numpy as jnp
from jax import lax
from jax.experimental import pallas as pl
from jax.experimental.pallas import tpu as pltpu
from jax.experimental.pallas import tpu_sc as plsc

N_TOK = 2048
NB_IN = 768
NB_OUT = 768
NB_EXPERTS = 8
NB_HIDDEN = 3072
TOP_K = 2

BM = 256
NT = (TOP_K * N_TOK) // BM + NB_EXPERTS
P_STATIC = NT * BM
SLEN = 2 * NT + NB_EXPERTS + 3

NW = 32
TPW = N_TOK // NW


def _router_body(x_ref, gnw_ref, gnb_ref, noise_ref,
                 w_ref, p0_ref, p1_ref, d0_ref, d1_ref, be_ref):
    x = x_ref[...]
    lg = jax.lax.dot_general(
        x, gnw_ref[...], (((1,), (1,)), ((), ())),
        preferred_element_type=jnp.float32,
    ) + gnb_ref[...]
    gate = lg[:, :NB_EXPERTS]
    nz = lg[:, NB_EXPERTS:]
    sp = jnp.maximum(nz, 0.0) + jnp.log1p(jnp.exp(-jnp.abs(nz)))
    logits = gate + noise_ref[...] * sp

    bi = jax.lax.broadcasted_iota(jnp.int32, (N_TOK, NB_EXPERTS), 1)
    v0 = jnp.max(logits, axis=1, keepdims=True)
    i0 = jnp.min(jnp.where(logits == v0, bi, NB_EXPERTS), axis=1, keepdims=True)
    masked = jnp.where(bi == i0, -jnp.inf, logits)
    v1 = jnp.max(masked, axis=1, keepdims=True)
    i1 = jnp.min(jnp.where(masked == v1, bi, NB_EXPERTS), axis=1, keepdims=True)
    e = jnp.exp(v1 - v0)
    p0 = 1.0 / (1.0 + e)
    p1 = e / (1.0 + e)
    w_ref[...] = jnp.where(bi == i0, p0, 0.0) + jnp.where(bi == i1, p1, 0.0)
    p0_ref[...] = p0
    p1_ref[...] = p1

    oh0 = (bi == i0).astype(jnp.float32)
    oh1 = (bi == i1).astype(jnp.float32)
    oh = jnp.concatenate([oh0, oh1], axis=1)
    c = oh
    s = 1
    while s < N_TOK:
        c = c + jnp.concatenate(
            [jnp.zeros((s, 2 * NB_EXPERTS), jnp.float32), c[:-s, :]], axis=0)
        s *= 2
    c0 = c[:, :NB_EXPERTS]
    c1 = c[:, NB_EXPERTS:]
    cnt0 = c0[N_TOK - 1:N_TOK, :]
    cnt = cnt0 + c1[N_TOK - 1:N_TOK, :]

    pc = jnp.floor((cnt + (BM - 1)) * (1.0 / BM)).astype(jnp.int32)
    pc = pc.astype(jnp.float32) * BM
    t = pc
    for sh in (1, 2, 4):
        t = t + jnp.concatenate(
            [jnp.zeros((1, sh), jnp.float32), t[:, :-sh]], axis=1)
    off = t - pc
    total = jnp.sum(pc, axis=1, keepdims=True)

    rank0 = jnp.sum((c0 - oh0) * oh0, axis=1, keepdims=True)
    rank1 = jnp.sum((c1 - oh1 + cnt0) * oh1, axis=1, keepdims=True)
    off0 = jnp.sum(off * oh0, axis=1, keepdims=True)
    off1 = jnp.sum(off * oh1, axis=1, keepdims=True)
    d0_ref[...] = (off0 + rank0).astype(jnp.int32)
    d1_ref[...] = (off1 + rank1).astype(jnp.int32)

    ends = off + pc
    bv = jax.lax.broadcasted_iota(
        jnp.int32, (NT, NB_EXPERTS), 0).astype(jnp.float32) * BM
    bv = jnp.minimum(bv, total - BM)
    blk = jnp.sum((ends <= bv).astype(jnp.float32), axis=1, keepdims=True)
    be = jnp.minimum(blk, NB_EXPERTS - 1).astype(jnp.int32)

    be_sh = jnp.concatenate(
        [jnp.full((1, 1), -1, jnp.int32), be[:-1, :]], axis=0)
    starts = (be != be_sh).astype(jnp.float32)
    g = starts
    sh = 1
    while sh < NT:
        g = g + jnp.concatenate(
            [jnp.zeros((sh, 1), jnp.float32), g[:-sh, :]], axis=0)
        sh *= 2
    gid = g.astype(jnp.int32) - 1

    nonempty = (cnt > 0.0).astype(jnp.float32)
    r = nonempty
    for sh in (1, 2, 4):
        r = r + jnp.concatenate(
            [jnp.zeros((1, sh), jnp.float32), r[:, :-sh]], axis=1)
    ngroups = jnp.sum(nonempty, axis=1, keepdims=True)
    giota = jax.lax.broadcasted_iota(
        jnp.int32, (NB_EXPERTS + 1, NB_EXPERTS), 0).astype(jnp.float32)
    eiota = jax.lax.broadcasted_iota(
        jnp.int32, (NB_EXPERTS + 1, NB_EXPERTS), 1).astype(jnp.float32)
    m = jnp.where((r - 1.0 == giota) & (nonempty > 0.0), 1.0, 0.0)
    ge = jnp.sum(m * eiota, axis=1, keepdims=True)

    be_ref[0:NT, :] = be
    be_ref[NT:2 * NT, :] = gid
    be_ref[2 * NT:2 * NT + 1, :] = (total * (1.0 / BM)).astype(jnp.int32)
    be_ref[2 * NT + 1:2 * NT + NB_EXPERTS + 2, :] = ge.astype(jnp.int32)
    be_ref[2 * NT + NB_EXPERTS + 2:2 * NT + NB_EXPERTS + 3, :] = (
        ngroups.astype(jnp.int32))


def _router(x, gate_w, gate_b, noise_w, noise_b, noise):
    gnw = jnp.concatenate([gate_w, noise_w], axis=0)
    gnb = jnp.concatenate([gate_b, noise_b], axis=0)[None, :]
    return pl.pallas_call(
        _router_body,
        out_shape=[
            jax.ShapeDtypeStruct((N_TOK, NB_EXPERTS), jnp.float32),
            jax.ShapeDtypeStruct((N_TOK, 1), jnp.float32),
            jax.ShapeDtypeStruct((N_TOK, 1), jnp.float32),
            jax.ShapeDtypeStruct((N_TOK, 1), jnp.int32),
            jax.ShapeDtypeStruct((N_TOK, 1), jnp.int32),
            jax.ShapeDtypeStruct((SLEN, 1), jnp.int32),
        ],
    )(x, gnw, gnb, noise)


def _sc_dispatch_body(x_hbm, d0_hbm, d1_hbm, out_hbm,
                      xrows_v, idx0_v, idx1_v, sem):
    w = lax.axis_index("s") * 2 + lax.axis_index("c")
    base = w * TPW
    pltpu.sync_copy(x_hbm.at[pl.ds(base, TPW)], xrows_v)
    pltpu.sync_copy(d0_hbm.at[pl.ds(base, TPW)], idx0_v)
    pltpu.sync_copy(d1_hbm.at[pl.ds(base, TPW)], idx1_v)
    pltpu.async_copy(xrows_v, out_hbm.at[idx0_v], sem).wait()
    pltpu.async_copy(xrows_v, out_hbm.at[idx1_v], sem).wait()


@functools.cache
def _get_sc_dispatch():
    return pl.kernel(
        _sc_dispatch_body,
        out_type=jax.ShapeDtypeStruct((P_STATIC, NB_IN), jnp.float32),
        mesh=plsc.VectorSubcoreMesh(core_axis_name="c", subcore_axis_name="s"),
        scratch_types=[
            pltpu.VMEM((TPW, NB_IN), jnp.float32),
            pltpu.VMEM((TPW,), jnp.int32),
            pltpu.VMEM((TPW,), jnp.int32),
            pltpu.SemaphoreType.DMA,
        ],
    )


HH = NB_HIDDEN // 2


def _gemm_body(s_ref, x_ref, b1_ref, b2_ref, w1_hbm, w2_hbm, y_ref,
               w1v, w2v, s1a, s2a, s1b, s2b):
    i = pl.program_id(0)
    na = s_ref[2 * NT]
    g = s_ref[NT + i]
    ng = s_ref[2 * NT + NB_EXPERTS + 2]

    def half_copies(gg, slot):
        e = s_ref[2 * NT + 1 + gg]
        return (
            pltpu.make_async_copy(
                w1_hbm.at[e, pl.ds(0, HH)],
                w1v.at[slot, pl.ds(0, HH)], s1a.at[slot]),
            pltpu.make_async_copy(
                w2_hbm.at[e, :, pl.ds(0, HH)],
                w2v.at[slot, :, pl.ds(0, HH)], s2a.at[slot]),
            pltpu.make_async_copy(
                w1_hbm.at[e, pl.ds(HH, HH)],
                w1v.at[slot, pl.ds(HH, HH)], s1b.at[slot]),
            pltpu.make_async_copy(
                w2_hbm.at[e, :, pl.ds(HH, HH)],
                w2v.at[slot, :, pl.ds(HH, HH)], s2b.at[slot]),
        )

    def start_copy(gg, slot):
        for c in half_copies(gg, slot):
            c.start()

    @pl.when(i == 0)
    def _prime():
        start_copy(0, 0)

        @pl.when(ng > 1)
        def _():
            start_copy(1, 1)

    prev_g = s_ref[NT + jnp.maximum(i - 1, 0)]
    first = jnp.logical_and(jnp.logical_or(i == 0, g != prev_g), i < na)
    slot = jax.lax.rem(g, 2)

    @pl.when(first)
    def _prefetch_next():
        @pl.when(jnp.logical_and(g >= 1, g + 1 < ng))
        def _():
            start_copy(g + 1, jax.lax.rem(g + 1, 2))

    @pl.when(i < na)
    def _compute():
        c1a, c2a, c1b, c2b = half_copies(g, slot)
        xb = x_ref[...].astype(jnp.bfloat16)

        @pl.when(first)
        def _w():
            c1a.wait()
        ha = jax.lax.dot_general(
            xb, w1v[slot, pl.ds(0, HH)].astype(jnp.bfloat16),
            (((1,), (1,)), ((), ())), preferred_element_type=jnp.float32)
        ha = jnp.maximum(ha + b1_ref[0][:, :HH], 0.0).astype(jnp.bfloat16)

        @pl.when(first)
        def _w2():
            c2a.wait()
        y = jax.lax.dot_general(
            ha, w2v[slot, :, pl.ds(0, HH)].astype(jnp.bfloat16),
            (((1,), (1,)), ((), ())), preferred_element_type=jnp.float32)

        @pl.when(first)
        def _w3():
            c1b.wait()
        hb = jax.lax.dot_general(
            xb, w1v[slot, pl.ds(HH, HH)].astype(jnp.bfloat16),
            (((1,), (1,)), ((), ())), preferred_element_type=jnp.float32)
        hb = jnp.maximum(hb + b1_ref[0][:, HH:], 0.0).astype(jnp.bfloat16)

        @pl.when(first)
        def _w4():
            c2b.wait()
        y = y + jax.lax.dot_general(
            hb, w2v[slot, :, pl.ds(HH, HH)].astype(jnp.bfloat16),
            (((1,), (1,)), ((), ())), preferred_element_type=jnp.float32)
        y_ref[...] = y + b2_ref[0]


def _gemm(blk_meta, dispatch, w1, b1, w2, b2):
    grid_spec = pltpu.PrefetchScalarGridSpec(
        num_scalar_prefetch=1,
        grid=(NT,),
        in_specs=[
            pl.BlockSpec((BM, NB_IN),
                         lambda i, s: (jnp.minimum(i, s[2 * NT] - 1), 0)),
            pl.BlockSpec((1, 1, NB_HIDDEN), lambda i, s: (s[i], 0, 0)),
            pl.BlockSpec((1, 1, NB_OUT), lambda i, s: (s[i], 0, 0)),
            pl.BlockSpec(memory_space=pltpu.HBM),
            pl.BlockSpec(memory_space=pltpu.HBM),
        ],
        out_specs=pl.BlockSpec((BM, NB_OUT),
                               lambda i, s: (jnp.minimum(i, s[2 * NT] - 1), 0)),
        scratch_shapes=[
            pltpu.VMEM((2, NB_HIDDEN, NB_IN), jnp.float32),
            pltpu.VMEM((2, NB_OUT, NB_HIDDEN), jnp.float32),
            pltpu.SemaphoreType.DMA((2,)),
            pltpu.SemaphoreType.DMA((2,)),
            pltpu.SemaphoreType.DMA((2,)),
            pltpu.SemaphoreType.DMA((2,)),
        ],
    )
    return pl.pallas_call(
        _gemm_body,
        grid_spec=grid_spec,
        out_shape=jax.ShapeDtypeStruct((P_STATIC, NB_OUT), jnp.float32),
        compiler_params=pltpu.CompilerParams(
            dimension_semantics=("arbitrary",),
            vmem_limit_bytes=64 * 1024 * 1024,
        ),
    )(blk_meta, dispatch, b1[:, None, :], b2[:, None, :], w1, w2)


def _sc_combine_body(y_hbm, d0_hbm, d1_hbm, p0_hbm, p1_hbm, out_hbm,
                     r0_v, r1_v, p0_v, p1_v, idx_v, sem):
    w = lax.axis_index("s") * 2 + lax.axis_index("c")
    base = w * TPW
    pltpu.sync_copy(d0_hbm.at[pl.ds(base, TPW)], idx_v)
    cp0 = pltpu.async_copy(y_hbm.at[idx_v], r0_v, sem)
    pltpu.sync_copy(p0_hbm.at[pl.ds(base, TPW)], p0_v)
    pltpu.sync_copy(p1_hbm.at[pl.ds(base, TPW)], p1_v)
    cp0.wait()
    pltpu.sync_copy(d1_hbm.at[pl.ds(base, TPW)], idx_v)
    pltpu.async_copy(y_hbm.at[idx_v], r1_v, sem).wait()

    def tok(t, carry):
        tvec = jnp.broadcast_to(t, (16,)).astype(jnp.int32)
        p0s = plsc.load_gather(p0_v, [tvec])
        p1s = plsc.load_gather(p1_v, [tvec])
        for c in range(NB_OUT // 16):
            sl = pl.ds(c * 16, 16)
            r0_v[t, sl] = p0s * r0_v[t, sl] + p1s * r1_v[t, sl]
        return carry

    lax.fori_loop(0, TPW, tok, 0)
    pltpu.sync_copy(r0_v, out_hbm.at[pl.ds(base, TPW)])


@functools.cache
def _get_sc_combine():
    return pl.kernel(
        _sc_combine_body,
        out_type=jax.ShapeDtypeStruct((N_TOK, NB_OUT), jnp.float32),
        mesh=plsc.VectorSubcoreMesh(core_axis_name="c", subcore_axis_name="s"),
        scratch_types=[
            pltpu.VMEM((TPW, NB_OUT), jnp.float32),
            pltpu.VMEM((TPW, NB_OUT), jnp.float32),
            pltpu.VMEM((TPW,), jnp.float32),
            pltpu.VMEM((TPW,), jnp.float32),
            pltpu.VMEM((TPW,), jnp.int32),
            pltpu.SemaphoreType.DMA,
        ],
        compiler_params=pltpu.CompilerParams(needs_layout_passes=False),
    )


def kernel(x, gate_w, gate_b, noise_w, noise_b, w1, b1, w2, b2, noise):
    weights, p0, p1, d0, d1, meta = _router(
        x, gate_w, gate_b, noise_w, noise_b, noise)
    blk_meta = meta.reshape(SLEN)
    d0f = d0.reshape(N_TOK)
    d1f = d1.reshape(N_TOK)
    dispatch = _get_sc_dispatch()(x, d0f, d1f)
    y = _gemm(blk_meta, dispatch, w1, b1, w2, b2)
    x_out = _get_sc_combine()(
        y, d0f, d1f, p0.reshape(N_TOK), p1.reshape(N_TOK))
    return (x_out, weights)


# speedup vs baseline: 1.1724x; 1.1724x over previous
import functools

import jax
import jax.numpy as jnp
from jax import lax
from jax.experimental import pallas as pl
from jax.experimental.pallas import tpu as pltpu
from jax.experimental.pallas import tpu_sc as plsc

N_TOK = 2048
NB_IN = 768
NB_OUT = 768
NB_EXPERTS = 8
NB_HIDDEN = 3072
TOP_K = 2

BM = 256
NT = (TOP_K * N_TOK) // BM + NB_EXPERTS
P_STATIC = NT * BM
SLEN = 2 * NT + NB_EXPERTS + 3

NW = 32
TPW = N_TOK // NW


def _router_body(x_ref, gnw_ref, gnb_ref, noise_ref,
                 w_ref, p0_ref, p1_ref, d0_ref, d1_ref, be_ref):
    x = x_ref[...]
    lg = jax.lax.dot_general(
        x, gnw_ref[...], (((1,), (1,)), ((), ())),
        preferred_element_type=jnp.float32,
    ) + gnb_ref[...]
    gate = lg[:, :NB_EXPERTS]
    nz = lg[:, NB_EXPERTS:]
    sp = jnp.maximum(nz, 0.0) + jnp.log1p(jnp.exp(-jnp.abs(nz)))
    logits = gate + noise_ref[...] * sp

    bi = jax.lax.broadcasted_iota(jnp.int32, (N_TOK, NB_EXPERTS), 1)
    v0 = jnp.max(logits, axis=1, keepdims=True)
    i0 = jnp.min(jnp.where(logits == v0, bi, NB_EXPERTS), axis=1, keepdims=True)
    masked = jnp.where(bi == i0, -jnp.inf, logits)
    v1 = jnp.max(masked, axis=1, keepdims=True)
    i1 = jnp.min(jnp.where(masked == v1, bi, NB_EXPERTS), axis=1, keepdims=True)
    e = jnp.exp(v1 - v0)
    p0 = 1.0 / (1.0 + e)
    p1 = e / (1.0 + e)
    w_ref[...] = jnp.where(bi == i0, p0, 0.0) + jnp.where(bi == i1, p1, 0.0)
    p0_ref[...] = p0
    p1_ref[...] = p1

    oh0 = (bi == i0).astype(jnp.float32)
    oh1 = (bi == i1).astype(jnp.float32)
    oh = jnp.concatenate([oh0, oh1], axis=1)
    c = oh
    s = 1
    while s < N_TOK:
        c = c + jnp.concatenate(
            [jnp.zeros((s, 2 * NB_EXPERTS), jnp.float32), c[:-s, :]], axis=0)
        s *= 2
    c0 = c[:, :NB_EXPERTS]
    c1 = c[:, NB_EXPERTS:]
    cnt0 = c0[N_TOK - 1:N_TOK, :]
    cnt = cnt0 + c1[N_TOK - 1:N_TOK, :]

    pc = jnp.floor((cnt + (BM - 1)) * (1.0 / BM)).astype(jnp.int32)
    pc = pc.astype(jnp.float32) * BM
    t = pc
    for sh in (1, 2, 4):
        t = t + jnp.concatenate(
            [jnp.zeros((1, sh), jnp.float32), t[:, :-sh]], axis=1)
    off = t - pc
    total = jnp.sum(pc, axis=1, keepdims=True)

    rank0 = jnp.sum((c0 - oh0) * oh0, axis=1, keepdims=True)
    rank1 = jnp.sum((c1 - oh1 + cnt0) * oh1, axis=1, keepdims=True)
    off0 = jnp.sum(off * oh0, axis=1, keepdims=True)
    off1 = jnp.sum(off * oh1, axis=1, keepdims=True)
    d0_ref[...] = (off0 + rank0).astype(jnp.int32)
    d1_ref[...] = (off1 + rank1).astype(jnp.int32)

    ends = off + pc
    bv = jax.lax.broadcasted_iota(
        jnp.int32, (NT, NB_EXPERTS), 0).astype(jnp.float32) * BM
    bv = jnp.minimum(bv, total - BM)
    blk = jnp.sum((ends <= bv).astype(jnp.float32), axis=1, keepdims=True)
    be = jnp.minimum(blk, NB_EXPERTS - 1).astype(jnp.int32)

    be_sh = jnp.concatenate(
        [jnp.full((1, 1), -1, jnp.int32), be[:-1, :]], axis=0)
    starts = (be != be_sh).astype(jnp.float32)
    g = starts
    sh = 1
    while sh < NT:
        g = g + jnp.concatenate(
            [jnp.zeros((sh, 1), jnp.float32), g[:-sh, :]], axis=0)
        sh *= 2
    gid = g.astype(jnp.int32) - 1

    nonempty = (cnt > 0.0).astype(jnp.float32)
    r = nonempty
    for sh in (1, 2, 4):
        r = r + jnp.concatenate(
            [jnp.zeros((1, sh), jnp.float32), r[:, :-sh]], axis=1)
    ngroups = jnp.sum(nonempty, axis=1, keepdims=True)
    giota = jax.lax.broadcasted_iota(
        jnp.int32, (NB_EXPERTS + 1, NB_EXPERTS), 0).astype(jnp.float32)
    eiota = jax.lax.broadcasted_iota(
        jnp.int32, (NB_EXPERTS + 1, NB_EXPERTS), 1).astype(jnp.float32)
    m = jnp.where((r - 1.0 == giota) & (nonempty > 0.0), 1.0, 0.0)
    ge = jnp.sum(m * eiota, axis=1, keepdims=True)

    be_ref[0:NT, :] = be
    be_ref[NT:2 * NT, :] = gid
    be_ref[2 * NT:2 * NT + 1, :] = (total * (1.0 / BM)).astype(jnp.int32)
    be_ref[2 * NT + 1:2 * NT + NB_EXPERTS + 2, :] = ge.astype(jnp.int32)
    be_ref[2 * NT + NB_EXPERTS + 2:2 * NT + NB_EXPERTS + 3, :] = (
        ngroups.astype(jnp.int32))


def _router(x, gate_w, gate_b, noise_w, noise_b, noise):
    gnw = jnp.concatenate([gate_w, noise_w], axis=0)
    gnb = jnp.concatenate([gate_b, noise_b], axis=0)[None, :]
    return pl.pallas_call(
        _router_body,
        out_shape=[
            jax.ShapeDtypeStruct((N_TOK, NB_EXPERTS), jnp.float32),
            jax.ShapeDtypeStruct((N_TOK, 1), jnp.float32),
            jax.ShapeDtypeStruct((N_TOK, 1), jnp.float32),
            jax.ShapeDtypeStruct((N_TOK, 1), jnp.int32),
            jax.ShapeDtypeStruct((N_TOK, 1), jnp.int32),
            jax.ShapeDtypeStruct((SLEN, 1), jnp.int32),
        ],
    )(x, gnw, gnb, noise)


def _sc_dispatch_body(x_hbm, d0_hbm, d1_hbm, out_hbm,
                      xrows_v, idx0_v, idx1_v, sem):
    w = lax.axis_index("s") * 2 + lax.axis_index("c")
    base = w * TPW
    pltpu.sync_copy(x_hbm.at[pl.ds(base, TPW)], xrows_v)
    pltpu.sync_copy(d0_hbm.at[pl.ds(base, TPW)], idx0_v)
    pltpu.sync_copy(d1_hbm.at[pl.ds(base, TPW)], idx1_v)
    pltpu.async_copy(xrows_v, out_hbm.at[idx0_v], sem).wait()
    pltpu.async_copy(xrows_v, out_hbm.at[idx1_v], sem).wait()


@functools.cache
def _get_sc_dispatch():
    return pl.kernel(
        _sc_dispatch_body,
        out_type=jax.ShapeDtypeStruct((P_STATIC, NB_IN), jnp.float32),
        mesh=plsc.VectorSubcoreMesh(core_axis_name="c", subcore_axis_name="s"),
        scratch_types=[
            pltpu.VMEM((TPW, NB_IN), jnp.float32),
            pltpu.VMEM((TPW,), jnp.int32),
            pltpu.VMEM((TPW,), jnp.int32),
            pltpu.SemaphoreType.DMA,
        ],
    )


def _gemm_body(s_ref, x_ref, b1_ref, b2_ref, w1_hbm, w2_hbm, y_ref,
               w1v, w2v, sem1, sem2):
    i = pl.program_id(0)
    na = s_ref[2 * NT]
    g = s_ref[NT + i]
    ng = s_ref[2 * NT + NB_EXPERTS + 2]

    def start_copy(gg, slot):
        e = s_ref[2 * NT + 1 + gg]
        pltpu.make_async_copy(w1_hbm.at[e], w1v.at[slot], sem1.at[slot]).start()
        pltpu.make_async_copy(w2_hbm.at[e], w2v.at[slot], sem2.at[slot]).start()

    @pl.when(i == 0)
    def _prime():
        start_copy(0, 0)

        @pl.when(ng > 1)
        def _():
            start_copy(1, 1)

    prev_g = s_ref[NT + jnp.maximum(i - 1, 0)]
    first = jnp.logical_and(jnp.logical_or(i == 0, g != prev_g), i < na)
    slot = jax.lax.rem(g, 2)

    @pl.when(first)
    def _wait_and_prefetch():
        e = s_ref[2 * NT + 1 + g]
        pltpu.make_async_copy(w1_hbm.at[e], w1v.at[slot], sem1.at[slot]).wait()
        pltpu.make_async_copy(w2_hbm.at[e], w2v.at[slot], sem2.at[slot]).wait()

        @pl.when(jnp.logical_and(g >= 1, g + 1 < ng))
        def _():
            start_copy(g + 1, jax.lax.rem(g + 1, 2))

    @pl.when(i < na)
    def _compute():
        xb = x_ref[...].astype(jnp.bfloat16)
        w1b = w1v[slot].astype(jnp.bfloat16)
        h = jax.lax.dot_general(
            xb, w1b, (((1,), (1,)), ((), ())),
            preferred_element_type=jnp.float32)
        h = jnp.maximum(h + b1_ref[0], 0.0)
        w2b = w2v[slot].astype(jnp.bfloat16)
        y = jax.lax.dot_general(
            h.astype(jnp.bfloat16), w2b, (((1,), (1,)), ((), ())),
            preferred_element_type=jnp.float32)
        y_ref[...] = y + b2_ref[0]


def _gemm(blk_meta, dispatch, w1, b1, w2, b2):
    grid_spec = pltpu.PrefetchScalarGridSpec(
        num_scalar_prefetch=1,
        grid=(NT,),
        in_specs=[
            pl.BlockSpec((BM, NB_IN),
                         lambda i, s: (jnp.minimum(i, s[2 * NT] - 1), 0)),
            pl.BlockSpec((1, 1, NB_HIDDEN), lambda i, s: (s[i], 0, 0)),
            pl.BlockSpec((1, 1, NB_OUT), lambda i, s: (s[i], 0, 0)),
            pl.BlockSpec(memory_space=pltpu.HBM),
            pl.BlockSpec(memory_space=pltpu.HBM),
        ],
        out_specs=pl.BlockSpec((BM, NB_OUT),
                               lambda i, s: (jnp.minimum(i, s[2 * NT] - 1), 0)),
        scratch_shapes=[
            pltpu.VMEM((2, NB_HIDDEN, NB_IN), jnp.float32),
            pltpu.VMEM((2, NB_OUT, NB_HIDDEN), jnp.float32),
            pltpu.SemaphoreType.DMA((2,)),
            pltpu.SemaphoreType.DMA((2,)),
        ],
    )
    return pl.pallas_call(
        _gemm_body,
        grid_spec=grid_spec,
        out_shape=jax.ShapeDtypeStruct((P_STATIC, NB_OUT), jnp.float32),
        compiler_params=pltpu.CompilerParams(
            dimension_semantics=("arbitrary",),
            vmem_limit_bytes=64 * 1024 * 1024,
        ),
    )(blk_meta, dispatch, b1[:, None, :], b2[:, None, :], w1, w2)


def _sc_combine_body(y_hbm, d0_hbm, d1_hbm, p0_hbm, p1_hbm, out_hbm,
                     r0_v, r1_v, p0_v, p1_v, idx_v, sem):
    w = lax.axis_index("s") * 2 + lax.axis_index("c")
    base = w * TPW
    pltpu.sync_copy(d0_hbm.at[pl.ds(base, TPW)], idx_v)
    cp0 = pltpu.async_copy(y_hbm.at[idx_v], r0_v, sem)
    pltpu.sync_copy(p0_hbm.at[pl.ds(base, TPW)], p0_v)
    pltpu.sync_copy(p1_hbm.at[pl.ds(base, TPW)], p1_v)
    cp0.wait()
    pltpu.sync_copy(d1_hbm.at[pl.ds(base, TPW)], idx_v)
    pltpu.async_copy(y_hbm.at[idx_v], r1_v, sem).wait()

    def tok(t, carry):
        tvec = jnp.broadcast_to(t, (16,)).astype(jnp.int32)
        p0s = plsc.load_gather(p0_v, [tvec])
        p1s = plsc.load_gather(p1_v, [tvec])
        for c in range(NB_OUT // 16):
            sl = pl.ds(c * 16, 16)
            r0_v[t, sl] = p0s * r0_v[t, sl] + p1s * r1_v[t, sl]
        return carry

    lax.fori_loop(0, TPW, tok, 0)
    pltpu.sync_copy(r0_v, out_hbm.at[pl.ds(base, TPW)])


@functools.cache
def _get_sc_combine():
    return pl.kernel(
        _sc_combine_body,
        out_type=jax.ShapeDtypeStruct((N_TOK, NB_OUT), jnp.float32),
        mesh=plsc.VectorSubcoreMesh(core_axis_name="c", subcore_axis_name="s"),
        scratch_types=[
            pltpu.VMEM((TPW, NB_OUT), jnp.float32),
            pltpu.VMEM((TPW, NB_OUT), jnp.float32),
            pltpu.VMEM((TPW,), jnp.float32),
            pltpu.VMEM((TPW,), jnp.float32),
            pltpu.VMEM((TPW,), jnp.int32),
            pltpu.SemaphoreType.DMA,
        ],
        compiler_params=pltpu.CompilerParams(needs_layout_passes=False),
    )


def kernel(x, gate_w, gate_b, noise_w, noise_b, w1, b1, w2, b2, noise):
    weights, p0, p1, d0, d1, meta = _router(
        x, gate_w, gate_b, noise_w, noise_b, noise)
    blk_meta = meta.reshape(SLEN)
    d0f = d0.reshape(N_TOK)
    d1f = d1.reshape(N_TOK)
    dispatch = _get_sc_dispatch()(x, d0f, d1f)
    y = _gemm(blk_meta, dispatch, w1, b1, w2, b2)
    x_out = _get_sc_combine()(
        y, d0f, d1f, p0.reshape(N_TOK), p1.reshape(N_TOK))
    return (x_out, weights)
